# baseline (speedup 1.0000x reference)
import functools

import jax
import jax.numpy as jnp
from jax import lax
from jax.experimental import pallas as pl
from jax.experimental.pallas import tpu as pltpu
from jax.experimental.pallas import tpu_sc as plsc

NC = 2
NS = 16
NW = NC * NS
LANES = 16
IDX_PER_DMA = 128
BCHUNK = 512
NBUF = 2


@functools.partial(jax.jit, static_argnames=("batch", "seq", "dim"))
def _embed(idx3, token_table, pos_t, *, batch, seq, dim):
    units_per_s = batch // BCHUNK
    n_units = seq * units_per_s
    units_per_w = n_units // NW
    dmas_per_unit = BCHUNK // IDX_PER_DMA
    idx_rows_per_w = units_per_w * dmas_per_unit
    tr_n = dim // 8
    tc_n = batch // 128
    tc_per_unit = BCHUNK // 128

    def body(idx_hbm, tok_hbm, pos_hbm, out_hbm, idx_v, pos_v,
             rows0, rows1, outt0, outt1, gsem, osem):
        rbufs = (rows0, rows1)
        obufs = (outt0, outt1)
        wid = lax.axis_index("s") * NC + lax.axis_index("c")

        pltpu.sync_copy(idx_hbm.at[wid], idx_v)
        pltpu.sync_copy(pos_hbm, pos_v.at[:, pl.ds(0, seq)])

        iota = lax.iota(jnp.int32, LANES)

        def unit_su(g):
            u = wid * units_per_w + g
            return u // units_per_s, lax.rem(u, units_per_s)

        def start_gather(g, b):
            for q in range(dmas_per_unit):
                pltpu.async_copy(
                    tok_hbm.at[idx_v.at[dmas_per_unit * g + q]],
                    rbufs[b].at[pl.ds(q * IDX_PER_DMA, IDX_PER_DMA)],
                    gsem.at[b])

        def wait_gather(g, b):
            for q in range(dmas_per_unit):
                pltpu.make_async_copy(
                    tok_hbm.at[idx_v.at[dmas_per_unit * g + q]],
                    rbufs[b].at[pl.ds(q * IDX_PER_DMA, IDX_PER_DMA)],
                    gsem.at[b]).wait()

        def out_slice(g):
            s, bc = unit_su(g)
            return out_hbm.at[s, :, pl.ds(bc * tc_per_unit, tc_per_unit),
                              :, :]

        def start_out(g, b):
            pltpu.async_copy(obufs[b], out_slice(g), osem.at[b])

        def wait_out(g, b):
            pltpu.make_async_copy(obufs[b], out_slice(g), osem.at[b]).wait()

        def transpose_add(g, b):
            s, _ = unit_su(g)
            rows = rbufs[b]
            outt = obufs[b]

            def dbody(d, carry):
                tr = d // 8
                r = lax.rem(d, 8)
                pvec = jnp.full((LANES,), pos_v[d, pl.ds(s, LANES)][0],
                                jnp.float32)
                cold = jnp.full((LANES,), d, jnp.int32)
                for tcl in range(tc_per_unit):
                    for c0 in range(0, 128, LANES):
                        rid = iota + (tcl * 128 + c0)
                        vals = plsc.load_gather(rows, [rid, cold])
                        outt[tr, tcl, r, pl.ds(c0, LANES)] = vals + pvec
                return carry

            lax.fori_loop(0, dim, dbody, 0)

        start_gather(0, 0)

        def gg_body(gg, carry):
            for b in range(NBUF):
                g = gg * NBUF + b
                nb = (b + 1) % NBUF

                @pl.when(g + 1 < units_per_w)
                def _():
                    start_gather(g + 1, nb)

                wait_gather(g, b)

                @pl.when(g >= NBUF)
                def _():
                    wait_out(g - NBUF, b)

                transpose_add(g, b)
                start_out(g, b)
            return carry

        lax.fori_loop(0, units_per_w // NBUF, gg_body, 0)
        for k in range(NBUF):
            g = units_per_w - NBUF + k
            wait_out(g, g % NBUF)

    grid_kernel = pl.kernel(
        body,
        out_type=jax.ShapeDtypeStruct((seq, tr_n, tc_n, 8, 128),
                                      jnp.float32),
        mesh=plsc.VectorSubcoreMesh(core_axis_name="c", subcore_axis_name="s"),
        scratch_types=[
            pltpu.VMEM((idx_rows_per_w, IDX_PER_DMA), jnp.int32),
            pltpu.VMEM((dim, 256), jnp.float32),
            pltpu.VMEM((BCHUNK, dim), jnp.float32),
            pltpu.VMEM((BCHUNK, dim), jnp.float32),
            pltpu.VMEM((tr_n, tc_per_unit, 8, 128), jnp.float32),
            pltpu.VMEM((tr_n, tc_per_unit, 8, 128), jnp.float32),
            pltpu.SemaphoreType.DMA((NBUF,)),
            pltpu.SemaphoreType.DMA((NBUF,)),
        ],
        compiler_params=pltpu.CompilerParams(use_tc_tiling_on_sc=False,
                                             needs_layout_passes=False),
    )
    return grid_kernel(idx3, token_table, pos_t)


def kernel(inputs, token_table, pos_table):
    batch, seq = inputs.shape
    _, dim = token_table.shape
    total = batch * seq
    rows_per_w = total // NW
    idx3 = inputs.astype(jnp.int32).T.reshape(
        NW, rows_per_w // IDX_PER_DMA, IDX_PER_DMA)
    pos_t = pos_table.T
    out5 = _embed(idx3, token_table, pos_t, batch=batch, seq=seq, dim=dim)
    return out5.transpose(2, 4, 0, 1, 3).reshape(batch, seq, dim)


# speedup vs baseline: 1.5526x; 1.5526x over previous
import functools

import jax
import jax.numpy as jnp
from jax import lax
from jax.experimental import pallas as pl
from jax.experimental.pallas import tpu as pltpu
from jax.experimental.pallas import tpu_sc as plsc

NC = 2
NS = 16
NW = NC * NS
LANES = 16
IDX_PER_DMA = 128
BCHUNK = 512
NBUF = 2


@functools.partial(jax.jit, static_argnames=("batch", "seq", "dim"))
def _embed(idx3, token_table, pos_t, *, batch, seq, dim):
    units_per_s = batch // BCHUNK
    n_units = seq * units_per_s
    units_per_w = n_units // NW
    dmas_per_unit = BCHUNK // IDX_PER_DMA
    idx_rows_per_w = units_per_w * dmas_per_unit
    tr_n = dim // 8
    tc_n = batch // 128
    tc_per_unit = BCHUNK // 128

    def body(idx_hbm, tok_hbm, pos_hbm, out_hbm, idx_v, pos_v,
             rows0, rows1, outt0, outt1, gsem, osem):
        rbufs = (rows0, rows1)
        obufs = (outt0, outt1)
        wid = lax.axis_index("s") * NC + lax.axis_index("c")

        pltpu.sync_copy(idx_hbm.at[wid], idx_v)
        pltpu.sync_copy(pos_hbm, pos_v.at[:, pl.ds(0, seq)])

        iota = lax.iota(jnp.int32, LANES)

        def unit_su(g):
            u = wid * units_per_w + g
            return u // units_per_s, lax.rem(u, units_per_s)

        def start_gather(g, b):
            for q in range(dmas_per_unit):
                pltpu.async_copy(
                    tok_hbm.at[idx_v.at[dmas_per_unit * g + q]],
                    rbufs[b].at[pl.ds(q * IDX_PER_DMA, IDX_PER_DMA)],
                    gsem.at[b])

        def wait_gather(g, b):
            for q in range(dmas_per_unit):
                pltpu.make_async_copy(
                    tok_hbm.at[idx_v.at[dmas_per_unit * g + q]],
                    rbufs[b].at[pl.ds(q * IDX_PER_DMA, IDX_PER_DMA)],
                    gsem.at[b]).wait()

        def out_slice(g):
            s, bc = unit_su(g)
            return out_hbm.at[s, :, pl.ds(bc * tc_per_unit, tc_per_unit),
                              :, :]

        def start_out(g, b):
            pltpu.async_copy(obufs[b], out_slice(g), osem.at[b])

        def wait_out(g, b):
            pltpu.make_async_copy(obufs[b], out_slice(g), osem.at[b]).wait()

        def transpose_add(g, b):
            s, _ = unit_su(g)
            rows = rbufs[b]
            outt = obufs[b]
            svec = jnp.full((LANES,), s, jnp.int32)

            def kbody(kk, carry):
                h = (kk // LANES) * LANES
                k = lax.rem(kk, LANES)
                dcol = lax.bitwise_and(k + iota, LANES - 1) + h
                tr_v = lax.shift_right_logical(dcol, 3)
                r_v = lax.bitwise_and(dcol, 7)
                pvec = plsc.load_gather(pos_v, [dcol, svec])
                for tcl in range(tc_per_unit):
                    tcl_v = jnp.full((LANES,), tcl, jnp.int32)
                    for c0 in range(0, 128, LANES):
                        rid = iota + (tcl * 128 + c0)
                        vals = plsc.load_gather(rows, [rid, dcol])
                        plsc.store_scatter(outt,
                                           [tr_v, tcl_v, r_v, iota + c0],
                                           vals + pvec)
                return carry

            lax.fori_loop(0, (dim // LANES) * LANES, kbody, 0)

        start_gather(0, 0)

        def gg_body(gg, carry):
            for b in range(NBUF):
                g = gg * NBUF + b
                nb = (b + 1) % NBUF

                @pl.when(g + 1 < units_per_w)
                def _():
                    start_gather(g + 1, nb)

                wait_gather(g, b)

                @pl.when(g >= NBUF)
                def _():
                    wait_out(g - NBUF, b)

                transpose_add(g, b)
                start_out(g, b)
            return carry

        lax.fori_loop(0, units_per_w // NBUF, gg_body, 0)
        for k in range(NBUF):
            g = units_per_w - NBUF + k
            wait_out(g, g % NBUF)

    grid_kernel = pl.kernel(
        body,
        out_type=jax.ShapeDtypeStruct((seq, tr_n, tc_n, 8, 128),
                                      jnp.float32),
        mesh=plsc.VectorSubcoreMesh(core_axis_name="c", subcore_axis_name="s"),
        scratch_types=[
            pltpu.VMEM((idx_rows_per_w, IDX_PER_DMA), jnp.int32),
            pltpu.VMEM((dim, 257), jnp.float32),
            pltpu.VMEM((BCHUNK, dim), jnp.float32),
            pltpu.VMEM((BCHUNK, dim), jnp.float32),
            pltpu.VMEM((tr_n, tc_per_unit, 8, 128), jnp.float32),
            pltpu.VMEM((tr_n, tc_per_unit, 8, 128), jnp.float32),
            pltpu.SemaphoreType.DMA((NBUF,)),
            pltpu.SemaphoreType.DMA((NBUF,)),
        ],
        compiler_params=pltpu.CompilerParams(use_tc_tiling_on_sc=False,
                                             needs_layout_passes=False),
    )
    return grid_kernel(idx3, token_table, pos_t)


def kernel(inputs, token_table, pos_table):
    batch, seq = inputs.shape
    _, dim = token_table.shape
    total = batch * seq
    rows_per_w = total // NW
    idx3 = inputs.astype(jnp.int32).T.reshape(
        NW, rows_per_w // IDX_PER_DMA, IDX_PER_DMA)
    pos_t = pos_table.T
    out5 = _embed(idx3, token_table, pos_t, batch=batch, seq=seq, dim=dim)
    return out5.transpose(2, 4, 0, 1, 3).reshape(batch, seq, dim)


# speedup vs baseline: 1.5782x; 1.0165x over previous
import functools

import jax
import jax.numpy as jnp
from jax import lax
from jax.experimental import pallas as pl
from jax.experimental.pallas import tpu as pltpu
from jax.experimental.pallas import tpu_sc as plsc

NC = 2
NS = 16
NW = NC * NS
LANES = 16
IDX_PER_DMA = 128
BCHUNK = 512
NBUF = 2


def _tc_detile_transpose(tok_t, *, dim, blk=2048):
    _, v = tok_t.shape
    nblk = v // blk

    def body(x_ref, o_ref):
        t = jnp.transpose(x_ref[...], (1, 0))
        t3 = t.reshape(blk // (128 // dim), 128 // dim, dim)
        o_ref[...] = jnp.concatenate(
            [t3[:, q, :] for q in range(128 // dim)], axis=-1)

    return pl.pallas_call(
        body,
        grid=(nblk,),
        in_specs=[pl.BlockSpec((dim, blk), lambda i: (0, i))],
        out_specs=pl.BlockSpec((blk * dim // 128, 128), lambda i: (i, 0)),
        out_shape=jax.ShapeDtypeStruct((v * dim // 128, 128), jnp.float32),
    )(tok_t)


@functools.partial(jax.jit, static_argnames=("batch", "seq", "dim"))
def _embed(idx3, token_table, pos_t, *, batch, seq, dim):
    units_per_s = batch // BCHUNK
    n_units = seq * units_per_s
    units_per_w = n_units // NW
    dmas_per_unit = BCHUNK // IDX_PER_DMA
    idx_rows_per_w = units_per_w * dmas_per_unit
    tr_n = dim // 8
    tc_n = batch // 128
    tc_per_unit = BCHUNK // 128

    def body(idx_hbm, tok_hbm, pos_hbm, out_hbm, idx_v, pos_v,
             rows0, rows1, outt0, outt1, gsem, osem):
        rbufs = (rows0, rows1)
        obufs = (outt0, outt1)
        wid = lax.axis_index("s") * NC + lax.axis_index("c")

        pltpu.sync_copy(idx_hbm.at[wid], idx_v)
        pltpu.sync_copy(pos_hbm, pos_v.at[:, pl.ds(0, seq)])

        iota = lax.iota(jnp.int32, LANES)

        def unit_su(g):
            u = wid * units_per_w + g
            return u // units_per_s, lax.rem(u, units_per_s)

        def start_gather(g, b):
            for q in range(dmas_per_unit):
                pltpu.async_copy(
                    tok_hbm.at[idx_v.at[dmas_per_unit * g + q]],
                    rbufs[b].at[pl.ds(q * IDX_PER_DMA, IDX_PER_DMA)],
                    gsem.at[b])

        def wait_gather(g, b):
            for q in range(dmas_per_unit):
                pltpu.make_async_copy(
                    tok_hbm.at[idx_v.at[dmas_per_unit * g + q]],
                    rbufs[b].at[pl.ds(q * IDX_PER_DMA, IDX_PER_DMA)],
                    gsem.at[b]).wait()

        def out_slice(g):
            s, bc = unit_su(g)
            return out_hbm.at[s, :, pl.ds(bc * tc_per_unit, tc_per_unit),
                              :, :]

        def start_out(g, b):
            pltpu.async_copy(obufs[b], out_slice(g), osem.at[b])

        def wait_out(g, b):
            pltpu.make_async_copy(obufs[b], out_slice(g), osem.at[b]).wait()

        def transpose_add(g, b):
            s, _ = unit_su(g)
            rows = rbufs[b]
            outt = obufs[b]
            svec = jnp.full((LANES,), s, jnp.int32)

            def kbody(kk, carry):
                h = (kk // LANES) * LANES
                k = lax.rem(kk, LANES)
                dcol = lax.bitwise_and(k + iota, LANES - 1) + h
                tr_v = lax.shift_right_logical(dcol, 3)
                r_v = lax.bitwise_and(dcol, 7)
                pvec = plsc.load_gather(pos_v, [dcol, svec])
                for tcl in range(tc_per_unit):
                    tcl_v = jnp.full((LANES,), tcl, jnp.int32)
                    for c0 in range(0, 128, LANES):
                        rid = iota + (tcl * 128 + c0)
                        vals = plsc.load_gather(rows, [rid, dcol])
                        plsc.store_scatter(outt,
                                           [tr_v, tcl_v, r_v, iota + c0],
                                           vals + pvec)
                return carry

            lax.fori_loop(0, (dim // LANES) * LANES, kbody, 0)

        start_gather(0, 0)

        def gg_body(gg, carry):
            for b in range(NBUF):
                g = gg * NBUF + b
                nb = (b + 1) % NBUF

                @pl.when(g + 1 < units_per_w)
                def _():
                    start_gather(g + 1, nb)

                wait_gather(g, b)

                @pl.when(g >= NBUF)
                def _():
                    wait_out(g - NBUF, b)

                transpose_add(g, b)
                start_out(g, b)
            return carry

        lax.fori_loop(0, units_per_w // NBUF, gg_body, 0)
        for k in range(NBUF):
            g = units_per_w - NBUF + k
            wait_out(g, g % NBUF)

    grid_kernel = pl.kernel(
        body,
        out_type=jax.ShapeDtypeStruct((seq, tr_n, tc_n, 8, 128),
                                      jnp.float32),
        mesh=plsc.VectorSubcoreMesh(core_axis_name="c", subcore_axis_name="s"),
        scratch_types=[
            pltpu.VMEM((idx_rows_per_w, IDX_PER_DMA), jnp.int32),
            pltpu.VMEM((dim, 257), jnp.float32),
            pltpu.VMEM((BCHUNK, dim), jnp.float32),
            pltpu.VMEM((BCHUNK, dim), jnp.float32),
            pltpu.VMEM((tr_n, tc_per_unit, 8, 128), jnp.float32),
            pltpu.VMEM((tr_n, tc_per_unit, 8, 128), jnp.float32),
            pltpu.SemaphoreType.DMA((NBUF,)),
            pltpu.SemaphoreType.DMA((NBUF,)),
        ],
        compiler_params=pltpu.CompilerParams(use_tc_tiling_on_sc=False,
                                             needs_layout_passes=False),
    )
    return grid_kernel(idx3, token_table, pos_t)


def kernel(inputs, token_table, pos_table):
    batch, seq = inputs.shape
    _, dim = token_table.shape
    total = batch * seq
    rows_per_w = total // NW
    idx3 = inputs.astype(jnp.int32).T.reshape(
        NW, rows_per_w // IDX_PER_DMA, IDX_PER_DMA)
    pos_t = pos_table.T
    vocab = token_table.shape[0]
    table_lin = _tc_detile_transpose(
        token_table.T, dim=dim).reshape(vocab, dim)
    out5 = _embed(idx3, table_lin, pos_t, batch=batch, seq=seq, dim=dim)
    return out5.transpose(2, 4, 0, 1, 3).reshape(batch, seq, dim)
